# Initial kernel scaffold; baseline (speedup 1.0000x reference)
import jax
import jax.numpy as jnp
from jax.experimental import pallas as pl


def kernel(x, edge_index, batch, W_in, b_in, Wq, bq, Wk, bk, Wv, bv, Wskip, bskip, gamma, beta, W_out, b_out):
    raise NotImplementedError("write your pallas kernel here")


# speedup vs baseline: 8.6075x; 8.6075x over previous
import dataclasses
import functools

import jax
import jax.numpy as jnp
from jax import lax
from jax.experimental import pallas as pl
from jax.experimental.pallas import tpu as pltpu
from jax.experimental.pallas import tpu_sc as plsc

N = 10000
E = 160000
DIN = 256
HID = 512
HEADS = 4
C = HID // HEADS
L = 8
DOUT = 256
B = 64

NTILES = 32
NPT = 320
ECHUNK = 128
EPAD = E + ECHUNK
RC = 32
INV_SQRT_C = 1.0 / (C ** 0.5)

_mesh = plsc.VectorSubcoreMesh(core_axis_name="c", subcore_axis_name="s")

_sc_params = pltpu.CompilerParams()
if "needs_layout_passes" in pltpu.CompilerParams.__dataclass_fields__:
    _sc_params = dataclasses.replace(_sc_params, needs_layout_passes=False)


def _mm_bias_body(x_ref, w_ref, b_ref, o_ref):
    o_ref[...] = jnp.dot(x_ref[...], w_ref[...],
                         preferred_element_type=jnp.float32) + b_ref[...]


def _mm_bias(x, w, b, bn=400):
    n, k = x.shape
    m = w.shape[1]
    return pl.pallas_call(
        _mm_bias_body,
        grid=(n // bn,),
        in_specs=[
            pl.BlockSpec((bn, k), lambda i: (i, 0)),
            pl.BlockSpec((k, m), lambda i: (0, 0)),
            pl.BlockSpec((1, m), lambda i: (0, 0)),
        ],
        out_specs=pl.BlockSpec((bn, m), lambda i: (i, 0)),
        out_shape=jax.ShapeDtypeStruct((n, m), jnp.float32),
    )(x, w, b.reshape(1, m))


def _qkvs_body(h_ref, w_ref, b_ref, oq_ref, ok_ref, ov_ref, os_ref):
    r = jnp.dot(h_ref[...], w_ref[...],
                preferred_element_type=jnp.float32) + b_ref[...]
    oq_ref[...] = r[:, 0 * HID:1 * HID]
    ok_ref[...] = r[:, 1 * HID:2 * HID]
    ov_ref[...] = r[:, 2 * HID:3 * HID]
    os_ref[...] = r[:, 3 * HID:4 * HID]


def _qkvs(h, wcat, bcat, bn=400):
    out = jax.ShapeDtypeStruct((N, HID), jnp.float32)
    return pl.pallas_call(
        _qkvs_body,
        grid=(N // bn,),
        in_specs=[
            pl.BlockSpec((bn, HID), lambda i: (i, 0)),
            pl.BlockSpec((HID, 4 * HID), lambda i: (0, 0)),
            pl.BlockSpec((1, 4 * HID), lambda i: (0, 0)),
        ],
        out_specs=[pl.BlockSpec((bn, HID), lambda i: (i, 0))] * 4,
        out_shape=[out, out, out, out],
    )(h, wcat, bcat.reshape(1, 4 * HID))


def _ln_body(h_ref, agg_ref, skip_ref, g_ref, b_ref, o_ref):
    t = h_ref[...] + agg_ref[...] + skip_ref[...]
    mu = jnp.mean(t, axis=1, keepdims=True)
    var = jnp.mean((t - mu) ** 2, axis=1, keepdims=True)
    o_ref[...] = (t - mu) * lax.rsqrt(var + 1e-5) * g_ref[...] + b_ref[...]


def _ln(h, agg, skip, g, b, bn=400):
    return pl.pallas_call(
        _ln_body,
        grid=(N // bn,),
        in_specs=[
            pl.BlockSpec((bn, HID), lambda i: (i, 0)),
            pl.BlockSpec((bn, HID), lambda i: (i, 0)),
            pl.BlockSpec((bn, HID), lambda i: (i, 0)),
            pl.BlockSpec((1, HID), lambda i: (0, 0)),
            pl.BlockSpec((1, HID), lambda i: (0, 0)),
        ],
        out_specs=pl.BlockSpec((bn, HID), lambda i: (i, 0)),
        out_shape=jax.ShapeDtypeStruct((N, HID), jnp.float32),
    )(h, agg, skip, g.reshape(1, HID), b.reshape(1, HID))


@functools.partial(
    pl.kernel,
    mesh=_mesh,
    out_type=jax.ShapeDtypeStruct((N, HID), jnp.float32),
    scratch_types=[
        pltpu.VMEM((NPT, C), jnp.float32),
        pltpu.VMEM((NPT, C), jnp.float32),
        pltpu.VMEM((ECHUNK, C), jnp.float32),
        pltpu.VMEM((ECHUNK, C), jnp.float32),
        pltpu.VMEM((ECHUNK,), jnp.int32),
        pltpu.VMEM((ECHUNK,), jnp.int32),
        pltpu.SMEM((NPT,), jnp.float32),
        pltpu.VMEM((1, 16), jnp.int32),
    ],
    compiler_params=_sc_params,
)
def _edge_kernel(q_hbm, kf_hbm, vf_hbm, srcs_hbm, dsts_hbm, bounds_hbm,
                 out_hbm, q_l, agg_l, k_c, v_c, si_c, d_c, den, bnd):
    wid = lax.axis_index("c") * 16 + lax.axis_index("s")
    pltpu.sync_copy(bounds_hbm.at[wid], bnd)
    bv = bnd[0, pl.ds(0, 16)]
    ea = pl.multiple_of(bv[0], 8)
    etot = bv[1]
    n0 = pl.multiple_of(bv[2], 8)
    nchunks = (etot + ECHUNK - 1) // ECHUNK
    zf = jnp.zeros((16,), jnp.float32)

    def head_body(h, carry_h):
        hc = pl.multiple_of(h * C, C)
        pltpu.sync_copy(q_hbm.at[pl.ds(n0, NPT), pl.ds(hc, C)], q_l)

        @pl.loop(0, NPT)
        def _(i):
            for c8 in range(C // 16):
                agg_l[i, pl.ds(c8 * 16, 16)] = zf

        @pl.loop(0, NPT)
        def _(i):
            den[i] = 0.0

        def chunk_body(ci, carry):
            base = pl.multiple_of(ea + ci * ECHUNK, 8)
            cnt = etot - ci * ECHUNK
            pltpu.sync_copy(srcs_hbm.at[pl.ds(base, ECHUNK)], si_c)
            pltpu.sync_copy(dsts_hbm.at[pl.ds(base, ECHUNK)], d_c)

            @pl.loop(0, ECHUNK, step=16)
            def _(i):
                si_c[pl.ds(i, 16)] = si_c[pl.ds(i, 16)] * HEADS + h

            pltpu.sync_copy(kf_hbm.at[si_c], k_c)
            pltpu.sync_copy(vf_hbm.at[si_c], v_c)

            def group_body(g, carry_g):
                e0 = g * 16
                dv = d_c[pl.ds(e0, 16)] - n0
                a_vec = zf
                for j in range(16):
                    dl = dv[j]
                    ok = jnp.logical_and(
                        jnp.logical_and(dl >= 0, dl < NPT), e0 + j < cnt)
                    dlc = jnp.clip(dl, 0, NPT - 1)
                    acc = zf
                    for c8 in range(C // 16):
                        s = pl.ds(c8 * 16, 16)
                        acc = acc + q_l[dlc, s] * k_c[e0 + j, s]
                    a = jnp.where(ok, jnp.sum(acc) * INV_SQRT_C,
                                  jnp.float32(-1e30))
                    a_vec = jnp.where(lax.iota(jnp.int32, 16) == j, a, a_vec)
                ex_vec = jnp.exp(a_vec)
                for j in range(16):
                    ex = ex_vec[j]
                    dlc = jnp.clip(dv[j], 0, NPT - 1)
                    den[dlc] = den[dlc] + ex
                    exv = jnp.full((16,), ex, jnp.float32)
                    for c8 in range(C // 16):
                        s = pl.ds(c8 * 16, 16)
                        plsc.addupdate(agg_l.at[dlc, s], exv * v_c[e0 + j, s])
                return carry_g

            lax.fori_loop(0, ECHUNK // 16, group_body, 0)
            return carry

        lax.fori_loop(0, nchunks, chunk_body, 0)

        @pl.loop(0, NPT)
        def _(i):
            dv = jnp.full((16,), den[i] + 1e-16, jnp.float32)
            rv = jnp.ones((16,), jnp.float32) / dv
            for c8 in range(C // 16):
                s = pl.ds(c8 * 16, 16)
                agg_l[i, s] = agg_l[i, s] * rv

        pltpu.sync_copy(agg_l, out_hbm.at[pl.ds(n0, NPT), pl.ds(hc, C)])
        return carry_h

    lax.fori_loop(0, HEADS, head_body, 0)


@functools.partial(
    pl.kernel,
    mesh=_mesh,
    out_type=jax.ShapeDtypeStruct((NTILES, 2, 2 * DOUT), jnp.float32),
    scratch_types=[
        pltpu.VMEM((RC, DOUT), jnp.float32),
        pltpu.VMEM((DOUT,), jnp.float32),
        pltpu.VMEM((DOUT,), jnp.float32),
        pltpu.VMEM((2, 2 * DOUT), jnp.float32),
        pltpu.VMEM((1, 16), jnp.int32),
    ],
    compiler_params=_sc_params,
)
def _pool_kernel(hx_hbm, bounds_hbm, out_hbm, rows, sbuf, mbuf, obuf, bnd):
    wid = lax.axis_index("c") * 16 + lax.axis_index("s")
    pltpu.sync_copy(bounds_hbm.at[wid], bnd)
    bv = bnd[0, pl.ds(0, 16)]
    for j in range(2):
        s = bv[2 * j]
        e = bv[2 * j + 1]
        cnt = e - s
        sa = pl.multiple_of((s // 8) * 8, 8)

        @pl.loop(0, DOUT, step=16)
        def _(i):
            sbuf[pl.ds(i, 16)] = jnp.zeros((16,), jnp.float32)
            mbuf[pl.ds(i, 16)] = jnp.full((16,), -3.0e38, jnp.float32)

        nchunks = (e - sa + RC - 1) // RC

        def chunk_body(ci, carry):
            st = sa + ci * RC
            stc = pl.multiple_of(jnp.minimum(st, N - RC), 8)
            pltpu.sync_copy(hx_hbm.at[pl.ds(stc, RC)], rows)
            lo = jnp.maximum(s, st) - stc
            hi = jnp.minimum(e, st + RC) - stc

            def row_body(i, _):
                for c16 in range(DOUT // 16):
                    sl = pl.ds(c16 * 16, 16)
                    vec = rows[i, sl]
                    sbuf[sl] = sbuf[sl] + vec
                    mbuf[sl] = jnp.maximum(mbuf[sl], vec)
                return 0

            lax.fori_loop(lo, hi, row_body, 0)
            return carry

        lax.fori_loop(0, nchunks, chunk_body, 0)

        cntf = jnp.maximum(cnt, 1).astype(jnp.float32)
        invv = jnp.ones((16,), jnp.float32) / jnp.full((16,), cntf, jnp.float32)
        nonempty = cnt > 0

        @pl.loop(0, DOUT, step=16)
        def _(i):
            sl = pl.ds(i, 16)
            obuf[j, sl] = sbuf[sl] * invv
            mv = jnp.where(nonempty, mbuf[sl], jnp.zeros((16,), jnp.float32))
            obuf[j, pl.ds(DOUT + i, 16)] = mv

    pltpu.sync_copy(obuf, out_hbm.at[wid])


def kernel(x, edge_index, batch, W_in, b_in, Wq, bq, Wk, bk, Wv, bv,
           Wskip, bskip, gamma, beta, W_out, b_out):
    src = edge_index[0]
    dst = edge_index[1]

    perm = jnp.argsort(dst)
    dst_s = dst[perm]
    src_s = src[perm]
    node_start = jnp.minimum(jnp.arange(NTILES, dtype=jnp.int32) * NPT, N - NPT)
    node_edge_start = jnp.searchsorted(dst_s, node_start).astype(jnp.int32)
    node_edge_end = jnp.searchsorted(dst_s, node_start + NPT).astype(jnp.int32)
    ea = (node_edge_start // 8) * 8
    etot = node_edge_end - ea
    ebounds = jnp.zeros((NTILES, 16), jnp.int32)
    ebounds = ebounds.at[:, 0].set(ea)
    ebounds = ebounds.at[:, 1].set(etot)
    ebounds = ebounds.at[:, 2].set(node_start)
    ebounds = ebounds.reshape(NTILES, 1, 16)
    src_p = jnp.concatenate([src_s, jnp.zeros((EPAD - E,), jnp.int32)])
    dst_p = jnp.concatenate([dst_s, jnp.zeros((EPAD - E,), jnp.int32)])

    bstart = jnp.searchsorted(batch, jnp.arange(B + 1, dtype=jnp.int32)).astype(jnp.int32)
    tix = jnp.arange(NTILES)
    pbounds = jnp.zeros((NTILES, 16), jnp.int32)
    pbounds = pbounds.at[:, 0].set(bstart[tix * 2])
    pbounds = pbounds.at[:, 1].set(bstart[tix * 2 + 1])
    pbounds = pbounds.at[:, 2].set(bstart[tix * 2 + 1])
    pbounds = pbounds.at[:, 3].set(bstart[tix * 2 + 2])
    pbounds = pbounds.reshape(NTILES, 1, 16)

    wcat = jnp.concatenate([Wq, Wk, Wv, Wskip], axis=2)
    bcat = jnp.concatenate([bq, bk, bv, bskip], axis=1)

    h = _mm_bias(x, W_in, b_in)
    for l in range(L):
        q, k, v, skip = _qkvs(h, wcat[l], bcat[l])
        kf = k.reshape(N * HEADS, C)
        vf = v.reshape(N * HEADS, C)
        agg = _edge_kernel(q, kf, vf, src_p, dst_p, ebounds)
        h = _ln(h, agg, skip, gamma[l], beta[l])
    hx = _mm_bias(h, W_out, b_out)
    return _pool_kernel(hx, pbounds).reshape(B, 2 * DOUT)


# speedup vs baseline: 12.5618x; 1.4594x over previous
import dataclasses
import functools

import jax
import jax.numpy as jnp
from jax import lax
from jax.experimental import pallas as pl
from jax.experimental.pallas import tpu as pltpu
from jax.experimental.pallas import tpu_sc as plsc

N = 10000
E = 160000
DIN = 256
HID = 512
HEADS = 4
C = HID // HEADS
L = 8
DOUT = 256
B = 64

NTILES = 32
NPT = 320
EC = 64
EPAD = E + 128
RC = 32
INV_SQRT_C = 1.0 / (C ** 0.5)

_mesh = plsc.VectorSubcoreMesh(core_axis_name="c", subcore_axis_name="s")

_sc_params = pltpu.CompilerParams()
if "needs_layout_passes" in pltpu.CompilerParams.__dataclass_fields__:
    _sc_params = dataclasses.replace(_sc_params, needs_layout_passes=False)


def _mm_bias_body(x_ref, w_ref, b_ref, o_ref):
    o_ref[...] = jnp.dot(x_ref[...], w_ref[...],
                         preferred_element_type=jnp.float32) + b_ref[...]


def _mm_bias(x, w, b, bn=400):
    n, k = x.shape
    m = w.shape[1]
    return pl.pallas_call(
        _mm_bias_body,
        grid=(n // bn,),
        in_specs=[
            pl.BlockSpec((bn, k), lambda i: (i, 0)),
            pl.BlockSpec((k, m), lambda i: (0, 0)),
            pl.BlockSpec((1, m), lambda i: (0, 0)),
        ],
        out_specs=pl.BlockSpec((bn, m), lambda i: (i, 0)),
        out_shape=jax.ShapeDtypeStruct((n, m), jnp.float32),
    )(x, w, b.reshape(1, m))


def _qkvs_body(h_ref, w_ref, b_ref, oq_ref, ok_ref, ov_ref, os_ref):
    r = jnp.dot(h_ref[...], w_ref[...],
                preferred_element_type=jnp.float32) + b_ref[...]
    oq_ref[...] = r[:, 0 * HID:1 * HID]
    ok_ref[...] = r[:, 1 * HID:2 * HID]
    ov_ref[...] = r[:, 2 * HID:3 * HID]
    os_ref[...] = r[:, 3 * HID:4 * HID]


def _qkvs(h, wcat, bcat, bn=400):
    out = jax.ShapeDtypeStruct((N, HID), jnp.float32)
    return pl.pallas_call(
        _qkvs_body,
        grid=(N // bn,),
        in_specs=[
            pl.BlockSpec((bn, HID), lambda i: (i, 0)),
            pl.BlockSpec((HID, 4 * HID), lambda i: (0, 0)),
            pl.BlockSpec((1, 4 * HID), lambda i: (0, 0)),
        ],
        out_specs=[pl.BlockSpec((bn, HID), lambda i: (i, 0))] * 4,
        out_shape=[out, out, out, out],
    )(h, wcat, bcat.reshape(1, 4 * HID))


def _ln_body(h_ref, agg_ref, skip_ref, g_ref, b_ref, o_ref):
    t = h_ref[...] + agg_ref[...] + skip_ref[...]
    mu = jnp.mean(t, axis=1, keepdims=True)
    var = jnp.mean((t - mu) ** 2, axis=1, keepdims=True)
    o_ref[...] = (t - mu) * lax.rsqrt(var + 1e-5) * g_ref[...] + b_ref[...]


def _ln(h, agg, skip, g, b, bn=400):
    return pl.pallas_call(
        _ln_body,
        grid=(N // bn,),
        in_specs=[
            pl.BlockSpec((bn, HID), lambda i: (i, 0)),
            pl.BlockSpec((bn, HID), lambda i: (i, 0)),
            pl.BlockSpec((bn, HID), lambda i: (i, 0)),
            pl.BlockSpec((1, HID), lambda i: (0, 0)),
            pl.BlockSpec((1, HID), lambda i: (0, 0)),
        ],
        out_specs=pl.BlockSpec((bn, HID), lambda i: (i, 0)),
        out_shape=jax.ShapeDtypeStruct((N, HID), jnp.float32),
    )(h, agg, skip, g.reshape(1, HID), b.reshape(1, HID))


@functools.partial(
    pl.kernel,
    mesh=_mesh,
    out_type=jax.ShapeDtypeStruct((N, HID), jnp.float32),
    scratch_types=[
        pltpu.VMEM((NPT, C), jnp.float32),
        pltpu.VMEM((NPT, C), jnp.float32),
        pltpu.VMEM((EC, C), jnp.float32),
        pltpu.VMEM((EC, C), jnp.float32),
        pltpu.VMEM((EC, C), jnp.float32),
        pltpu.VMEM((EC, C), jnp.float32),
        pltpu.VMEM((EC,), jnp.int32),
        pltpu.VMEM((EC,), jnp.int32),
        pltpu.VMEM((EC,), jnp.int32),
        pltpu.VMEM((EC,), jnp.int32),
        pltpu.VMEM((256,), jnp.float32),
        pltpu.SMEM((NPT,), jnp.float32),
        pltpu.VMEM((1, 16), jnp.int32),
        pltpu.SemaphoreType.DMA,
        pltpu.SemaphoreType.DMA,
        pltpu.SemaphoreType.DMA,
        pltpu.SemaphoreType.DMA,
    ],
    compiler_params=_sc_params,
)
def _edge_kernel(q_hbm, kf_hbm, vf_hbm, srcs_hbm, dsts_hbm, bounds_hbm,
                 out_hbm, q_l, agg_l, k0, k1, v0, v1, si0, si1, d0, d1,
                 p_buf, den, bnd, semi0, semi1, semg0, semg1):
    ks = (k0, k1)
    vs = (v0, v1)
    sis = (si0, si1)
    dds = (d0, d1)
    semi = (semi0, semi1)
    semg = (semg0, semg1)

    wid = lax.axis_index("c") * 16 + lax.axis_index("s")
    pltpu.sync_copy(bounds_hbm.at[wid], bnd)
    bv = bnd[0, pl.ds(0, 16)]
    ea = pl.multiple_of(bv[0], 8)
    etot = bv[1]
    n0 = pl.multiple_of(bv[2], 8)
    nchunks = (etot + EC - 1) // EC
    npairs = jnp.maximum((nchunks + 1) // 2, 1)
    zf = jnp.zeros((16,), jnp.float32)

    def cbase(ci):
        return pl.multiple_of(jnp.minimum(ea + ci * EC, EPAD - EC), 8)

    def fire_idx(ci, q):
        b = cbase(ci)
        pltpu.async_copy(srcs_hbm.at[pl.ds(b, EC)], sis[q], semi[q])
        pltpu.async_copy(dsts_hbm.at[pl.ds(b, EC)], dds[q], semi[q])

    def wait_idx(ci, q):
        b = cbase(ci)
        pltpu.make_async_copy(srcs_hbm.at[pl.ds(b, EC)], sis[q], semi[q]).wait()
        pltpu.make_async_copy(dsts_hbm.at[pl.ds(b, EC)], dds[q], semi[q]).wait()

    def fire_gather(q):
        pltpu.async_copy(kf_hbm.at[sis[q]], ks[q], semg[q])
        pltpu.async_copy(vf_hbm.at[sis[q]], vs[q], semg[q])

    def wait_gather(q):
        pltpu.make_async_copy(kf_hbm.at[sis[q]], ks[q], semg[q]).wait()
        pltpu.make_async_copy(vf_hbm.at[sis[q]], vs[q], semg[q]).wait()

    def head_body(h, carry_h):
        hc = pl.multiple_of(h * C, C)
        pltpu.sync_copy(q_hbm.at[pl.ds(n0, NPT), pl.ds(hc, C)], q_l)

        @pl.loop(0, NPT)
        def _(i):
            for c8 in range(C // 16):
                agg_l[i, pl.ds(c8 * 16, 16)] = zf

        @pl.loop(0, NPT)
        def _(i):
            den[i] = 0.0

        def transform(q):
            @pl.loop(0, EC, step=16)
            def _(i):
                sis[q][pl.ds(i, 16)] = sis[q][pl.ds(i, 16)] * HEADS + h

        def compute(ci, p):
            cnt = etot - ci * EC
            bidx = lax.iota(jnp.int32, 16) * 16

            def group_body(g, carry_g):
                e0 = g * 16
                dv = dds[p][pl.ds(e0, 16)] - n0
                okv = jnp.logical_and(
                    jnp.logical_and(dv >= 0, dv < NPT),
                    lax.iota(jnp.int32, 16) + e0 < cnt)
                dvc = jnp.clip(dv, 0, NPT - 1)
                for j in range(16):
                    dl = dvc[j]
                    acc = q_l[dl, pl.ds(0, 16)] * ks[p][e0 + j, pl.ds(0, 16)]
                    for c8 in range(1, C // 16):
                        s = pl.ds(c8 * 16, 16)
                        acc = acc + q_l[dl, s] * ks[p][e0 + j, s]
                    p_buf[pl.ds(j * 16, 16)] = acc
                sv = plsc.load_gather(p_buf, [bidx])
                for c in range(1, 16):
                    sv = sv + plsc.load_gather(p_buf, [bidx + c])
                a_vec = jnp.where(okv, sv * INV_SQRT_C,
                                  jnp.full((16,), -1e30, jnp.float32))
                ex_vec = jnp.exp(a_vec)
                for j in range(16):
                    ex = ex_vec[j]
                    dl = dvc[j]
                    den[dl] = den[dl] + ex
                    exv = jnp.full((16,), ex, jnp.float32)
                    for c8 in range(C // 16):
                        s = pl.ds(c8 * 16, 16)
                        plsc.addupdate(agg_l.at[dl, s], exv * vs[p][e0 + j, s])
                return carry_g

            lax.fori_loop(0, EC // 16, group_body, 0)

        fire_idx(0, 0)
        wait_idx(0, 0)
        transform(0)
        fire_gather(0)
        fire_idx(1, 1)

        def pair_body(pair, carry):
            for pp in range(2):
                ci = pair * 2 + pp
                wait_gather(pp)
                wait_idx(ci + 1, 1 - pp)
                transform(1 - pp)
                fire_gather(1 - pp)
                compute(ci, pp)
                fire_idx(ci + 2, pp)
            return carry

        lax.fori_loop(0, npairs, pair_body, 0)
        tc = npairs * 2
        wait_gather(0)
        wait_idx(tc + 1, 1)

        @pl.loop(0, NPT)
        def _(i):
            dv = jnp.full((16,), den[i] + 1e-16, jnp.float32)
            rv = jnp.ones((16,), jnp.float32) / dv
            for c8 in range(C // 16):
                s = pl.ds(c8 * 16, 16)
                agg_l[i, s] = agg_l[i, s] * rv

        pltpu.sync_copy(agg_l, out_hbm.at[pl.ds(n0, NPT), pl.ds(hc, C)])
        return carry_h

    lax.fori_loop(0, HEADS, head_body, 0)


@functools.partial(
    pl.kernel,
    mesh=_mesh,
    out_type=jax.ShapeDtypeStruct((NTILES, 2, 2 * DOUT), jnp.float32),
    scratch_types=[
        pltpu.VMEM((RC, DOUT), jnp.float32),
        pltpu.VMEM((DOUT,), jnp.float32),
        pltpu.VMEM((DOUT,), jnp.float32),
        pltpu.VMEM((2, 2 * DOUT), jnp.float32),
        pltpu.VMEM((1, 16), jnp.int32),
    ],
    compiler_params=_sc_params,
)
def _pool_kernel(hx_hbm, bounds_hbm, out_hbm, rows, sbuf, mbuf, obuf, bnd):
    wid = lax.axis_index("c") * 16 + lax.axis_index("s")
    pltpu.sync_copy(bounds_hbm.at[wid], bnd)
    bv = bnd[0, pl.ds(0, 16)]
    for j in range(2):
        s = bv[2 * j]
        e = bv[2 * j + 1]
        cnt = e - s
        sa = pl.multiple_of((s // 8) * 8, 8)

        @pl.loop(0, DOUT, step=16)
        def _(i):
            sbuf[pl.ds(i, 16)] = jnp.zeros((16,), jnp.float32)
            mbuf[pl.ds(i, 16)] = jnp.full((16,), -3.0e38, jnp.float32)

        nchunks = (e - sa + RC - 1) // RC

        def chunk_body(ci, carry):
            st = sa + ci * RC
            stc = pl.multiple_of(jnp.minimum(st, N - RC), 8)
            pltpu.sync_copy(hx_hbm.at[pl.ds(stc, RC)], rows)
            lo = jnp.maximum(s, st) - stc
            hi = jnp.minimum(e, st + RC) - stc

            def row_body(i, _):
                for c16 in range(DOUT // 16):
                    sl = pl.ds(c16 * 16, 16)
                    vec = rows[i, sl]
                    sbuf[sl] = sbuf[sl] + vec
                    mbuf[sl] = jnp.maximum(mbuf[sl], vec)
                return 0

            lax.fori_loop(lo, hi, row_body, 0)
            return carry

        lax.fori_loop(0, nchunks, chunk_body, 0)

        cntf = jnp.maximum(cnt, 1).astype(jnp.float32)
        invv = jnp.ones((16,), jnp.float32) / jnp.full((16,), cntf, jnp.float32)
        nonempty = cnt > 0

        @pl.loop(0, DOUT, step=16)
        def _(i):
            sl = pl.ds(i, 16)
            obuf[j, sl] = sbuf[sl] * invv
            mv = jnp.where(nonempty, mbuf[sl], jnp.zeros((16,), jnp.float32))
            obuf[j, pl.ds(DOUT + i, 16)] = mv

    pltpu.sync_copy(obuf, out_hbm.at[wid])


def kernel(x, edge_index, batch, W_in, b_in, Wq, bq, Wk, bk, Wv, bv,
           Wskip, bskip, gamma, beta, W_out, b_out):
    src = edge_index[0]
    dst = edge_index[1]

    perm = jnp.argsort(dst)
    dst_s = dst[perm]
    src_s = src[perm]
    node_start = jnp.minimum(jnp.arange(NTILES, dtype=jnp.int32) * NPT, N - NPT)
    node_edge_start = jnp.searchsorted(dst_s, node_start).astype(jnp.int32)
    node_edge_end = jnp.searchsorted(dst_s, node_start + NPT).astype(jnp.int32)
    ea = (node_edge_start // 8) * 8
    etot = node_edge_end - ea
    ebounds = jnp.zeros((NTILES, 16), jnp.int32)
    ebounds = ebounds.at[:, 0].set(ea)
    ebounds = ebounds.at[:, 1].set(etot)
    ebounds = ebounds.at[:, 2].set(node_start)
    ebounds = ebounds.reshape(NTILES, 1, 16)
    src_p = jnp.concatenate([src_s, jnp.zeros((EPAD - E,), jnp.int32)])
    dst_p = jnp.concatenate([dst_s, jnp.zeros((EPAD - E,), jnp.int32)])

    bstart = jnp.searchsorted(batch, jnp.arange(B + 1, dtype=jnp.int32)).astype(jnp.int32)
    tix = jnp.arange(NTILES)
    pbounds = jnp.zeros((NTILES, 16), jnp.int32)
    pbounds = pbounds.at[:, 0].set(bstart[tix * 2])
    pbounds = pbounds.at[:, 1].set(bstart[tix * 2 + 1])
    pbounds = pbounds.at[:, 2].set(bstart[tix * 2 + 1])
    pbounds = pbounds.at[:, 3].set(bstart[tix * 2 + 2])
    pbounds = pbounds.reshape(NTILES, 1, 16)

    wcat = jnp.concatenate([Wq, Wk, Wv, Wskip], axis=2)
    bcat = jnp.concatenate([bq, bk, bv, bskip], axis=1)

    h = _mm_bias(x, W_in, b_in)
    for l in range(L):
        q, k, v, skip = _qkvs(h, wcat[l], bcat[l])
        kf = k.reshape(N * HEADS, C)
        vf = v.reshape(N * HEADS, C)
        agg = _edge_kernel(q, kf, vf, src_p, dst_p, ebounds)
        h = _ln(h, agg, skip, gamma[l], beta[l])
    hx = _mm_bias(h, W_out, b_out)
    return _pool_kernel(hx, pbounds).reshape(B, 2 * DOUT)
